# baseline (speedup 1.0000x reference)
import functools

import numpy as np

import jax
import jax.numpy as jnp
from jax import lax
from jax.experimental import pallas as pl
from jax.experimental.pallas import tpu as pltpu
from jax.experimental.pallas import tpu_sc as plsc

_N = 10000
_E = 320000
_D = 128
_DE = 16
_H = 128
_G = 16
_ITERS = 5

_BLK = 2000
_EBLK = 2000
_K = 40
_NTILES = 32
_EPT = _E // _NTILES
_NCH = _EPT // _K
_SCN = 10240
_RPT = _SCN // 16


def _f32dot(a, b):
    return jnp.dot(a, b, preferred_element_type=jnp.float32)


def _pack_cols():
    lo = np.empty(_H // 2, np.int32)
    hi = np.empty(_H // 2, np.int32)
    for g in range(_H // 32):
        for l in range(16):
            lo[16 * g + l] = 32 * g + l
            hi[16 * g + l] = 32 * g + 16 + l
    return np.concatenate([lo, hi])


_PKMAP = _pack_cols()


def _pack_bf16_pairs(xf):
    lo = xf[:, :_H // 2].astype(jnp.bfloat16).astype(jnp.float32)
    hi = xf[:, _H // 2:].astype(jnp.bfloat16).astype(jnp.float32)
    lob = lax.bitcast_convert_type(lo, jnp.int32)
    hib = lax.bitcast_convert_type(hi, jnp.int32)
    return (hib & jnp.int32(-65536)) | lax.shift_right_logical(lob, 16)


def _pre_body(x_ref, we_ref, be_ref, ws_ref, wd_ref, wax_ref, ba_ref,
              h0_ref, a_ref, b_ref, sx_ref):
    x = x_ref[...]
    h0 = _f32dot(x, we_ref[...]) + be_ref[...]
    h0_ref[...] = h0
    a_ref[...] = _f32dot(h0, ws_ref[...])
    b_ref[...] = _f32dot(h0, wd_ref[...])
    sx_ref[...] = _f32dot(x, wax_ref[...]) + ba_ref[...]


def _pre(x, W_emb, be, Ws, Wd, Wax, ba):
    n_blocks = _N // _BLK
    return pl.pallas_call(
        _pre_body,
        grid=(n_blocks,),
        in_specs=[
            pl.BlockSpec((_BLK, _D), lambda i: (i, 0)),
            pl.BlockSpec((_D, _H), lambda i: (0, 0)),
            pl.BlockSpec((1, _H), lambda i: (0, 0)),
            pl.BlockSpec((_D, _H), lambda i: (0, 0)),
            pl.BlockSpec((_D, _H), lambda i: (0, 0)),
            pl.BlockSpec((_D, 1), lambda i: (0, 0)),
            pl.BlockSpec((1, 1), lambda i: (0, 0)),
        ],
        out_specs=[
            pl.BlockSpec((_BLK, _H), lambda i: (i, 0)),
            pl.BlockSpec((_BLK, _H), lambda i: (i, 0)),
            pl.BlockSpec((_BLK, _H), lambda i: (i, 0)),
            pl.BlockSpec((_BLK, 1), lambda i: (i, 0)),
        ],
        out_shape=[
            jax.ShapeDtypeStruct((_N, _H), jnp.float32),
            jax.ShapeDtypeStruct((_N, _H), jnp.float32),
            jax.ShapeDtypeStruct((_N, _H), jnp.float32),
            jax.ShapeDtypeStruct((_N, 1), jnp.float32),
        ],
    )(x, W_emb, be, Ws, Wd, Wax, ba)


def _edgec_body(ea2_ref, we_ref, bm_ref, c_ref):
    ea2 = ea2_ref[...]
    x1 = _f32dot(ea2[:, :_DE], we_ref[...]) + bm_ref[...]
    x2 = _f32dot(ea2[:, _DE:], we_ref[...]) + bm_ref[...]
    c_ref[...] = jnp.concatenate(
        [_pack_bf16_pairs(x1), _pack_bf16_pairs(x2)], axis=1)


def _edgec(edge_attr, We, bm):
    ea2 = edge_attr.reshape(_E // 2, 2 * _DE)
    return pl.pallas_call(
        _edgec_body,
        grid=(_E // 2 // _EBLK,),
        in_specs=[
            pl.BlockSpec((_EBLK, 2 * _DE), lambda i: (i, 0)),
            pl.BlockSpec((_DE, _H), lambda i: (0, 0)),
            pl.BlockSpec((1, _H), lambda i: (0, 0)),
        ],
        out_specs=pl.BlockSpec((_EBLK, _H), lambda i: (i, 0)),
        out_shape=jax.ShapeDtypeStruct((_E // 2, _H), jnp.int32),
    )(ea2, We, bm)


def _sc_edge_body(a_hbm, b_hbm, c_hbm, src_hbm, dst_hbm, out_hbm,
                  idxs0, idxs1, idxd0, idxd1, bufa0, bufa1, bufb0, bufb1,
                  bufc0, bufc1, shared,
                  sema0, sema1, semb0, semb1, semc0, semc1,
                  semis0, semis1, semid0, semid1):
    c = lax.axis_index("c")
    s = lax.axis_index("s")
    wid = c * 16 + s
    idxs = (idxs0, idxs1)
    idxd = (idxd0, idxd1)
    bufa = (bufa0, bufa1)
    bufb = (bufb0, bufb1)
    bufc = (bufc0, bufc1)
    sema = (sema0, sema1)
    semb = (semb0, semb1)
    semc = (semc0, semc1)
    semis = (semis0, semis1)
    semid = (semid0, semid1)

    zero16 = jnp.zeros((16,), jnp.float32)
    himask = jnp.int32(-65536)

    def zrow(r, carry):
        for jj in range(8):
            bufa0[r, pl.ds(jj * 16, 16)] = zero16
        return carry

    lax.fori_loop(0, _K, zrow, 0)
    row0 = s * _RPT
    for t in range(_RPT // _K):
        pltpu.sync_copy(bufa0, shared.at[pl.ds(row0 + t * _K, _K)])
    plsc.subcore_barrier()

    ebase = wid * _EPT

    def idx_issue(p, j):
        pltpu.async_copy(src_hbm.at[pl.ds(ebase + j * _K, _K)], idxs[p],
                         semis[p])
        pltpu.async_copy(dst_hbm.at[pl.ds(ebase + j * _K, _K)], idxd[p],
                         semid[p])

    def idx_wait(p, j):
        pltpu.make_async_copy(src_hbm.at[pl.ds(ebase + j * _K, _K)], idxs[p],
                              semis[p]).wait()
        pltpu.make_async_copy(dst_hbm.at[pl.ds(ebase + j * _K, _K)], idxd[p],
                              semid[p]).wait()

    def gather_issue(p, j):
        pltpu.async_copy(a_hbm.at[idxs[p]], bufa[p], sema[p])
        pltpu.async_copy(b_hbm.at[idxd[p]], bufb[p], semb[p])
        pltpu.async_copy(c_hbm.at[wid * _NCH + j], bufc[p], semc[p])

    def gather_wait(p, j):
        pltpu.make_async_copy(a_hbm.at[idxs[p]], bufa[p], sema[p]).wait()
        pltpu.make_async_copy(b_hbm.at[idxd[p]], bufb[p], semb[p]).wait()
        pltpu.make_async_copy(c_hbm.at[wid * _NCH + j], bufc[p],
                              semc[p]).wait()

    def compute(p):
        a, b, cc = bufa[p], bufb[p], bufc[p]

        def rowpair(t, rc):
            for half in range(2):
                r = 2 * t + half
                for g in range(4):
                    vc = cc[t, pl.ds(half * 64 + g * 16, 16)]
                    clo = lax.bitcast_convert_type(vc << 16, jnp.float32)
                    chi = lax.bitcast_convert_type(vc & himask, jnp.float32)
                    sl_lo = pl.ds(g * 32, 16)
                    sl_hi = pl.ds(g * 32 + 16, 16)
                    a[r, sl_lo] = jnp.maximum(
                        a[r, sl_lo] + b[r, sl_lo] + clo, zero16)
                    a[r, sl_hi] = jnp.maximum(
                        a[r, sl_hi] + b[r, sl_hi] + chi, zero16)
            return rc

        lax.fori_loop(0, _K // 2, rowpair, 0)

    def visit(p, j):
        q = 1 - p
        gather_wait(p, j)

        @pl.when(j + 1 < _NCH)
        def _():
            idx_wait(q, j + 1)
            gather_issue(q, j + 1)

        compute(p)
        pltpu.sync_copy(bufa[p], shared.at[idxd[p]], add=True)

        @pl.when(j + 2 < _NCH)
        def _():
            idx_issue(p, j + 2)

    idx_issue(0, 0)
    idx_issue(1, 1)
    idx_wait(0, 0)
    gather_issue(0, 0)

    def pair(t, carry):
        j = 2 * t
        visit(0, j)
        visit(1, j + 1)
        return carry

    lax.fori_loop(0, _NCH // 2, pair, 0)
    if _NCH % 2:
        visit(0, _NCH - 1)
    plsc.subcore_barrier()

    for t in range(_RPT // _K):
        pltpu.sync_copy(shared.at[pl.ds(row0 + t * _K, _K)],
                        out_hbm.at[c, pl.ds(row0 + t * _K, _K)])


def _sc_edge(A, B, C, src, dst):
    mesh = plsc.VectorSubcoreMesh(core_axis_name="c", subcore_axis_name="s")
    fn = functools.partial(
        pl.kernel,
        out_type=jax.ShapeDtypeStruct((2, _SCN, _H), jnp.float32),
        mesh=mesh,
        scratch_types=[
            pltpu.VMEM((_K,), jnp.int32),
            pltpu.VMEM((_K,), jnp.int32),
            pltpu.VMEM((_K,), jnp.int32),
            pltpu.VMEM((_K,), jnp.int32),
            pltpu.VMEM((_K, _H), jnp.float32),
            pltpu.VMEM((_K, _H), jnp.float32),
            pltpu.VMEM((_K, _H), jnp.float32),
            pltpu.VMEM((_K, _H), jnp.float32),
            pltpu.VMEM((_K // 2, _H), jnp.int32),
            pltpu.VMEM((_K // 2, _H), jnp.int32),
            pltpu.VMEM_SHARED((_SCN, _H), jnp.float32),
        ] + [pltpu.SemaphoreType.DMA] * 10,
    )(_sc_edge_body)
    return fn(A, B, C.reshape(_E // _K, _K // 2, _H), src, dst)


def _gred_body(a0_ref, a1_ref, bt_ref, g_ref):
    @pl.when(pl.program_id(0) == 0)
    def _():
        g_ref[...] = jnp.zeros_like(g_ref)

    hc = a0_ref[0] + a1_ref[0]
    m = (bt_ref[...] == lax.broadcasted_iota(jnp.int32, (_BLK, _G), 1)
         ).astype(jnp.float32)
    g_ref[...] += lax.dot_general(m, hc, (((0,), (0,)), ((), ())),
                                  preferred_element_type=jnp.float32)


def _gred(aggp, bt):
    return pl.pallas_call(
        _gred_body,
        grid=(_N // _BLK,),
        in_specs=[
            pl.BlockSpec((1, _BLK, _H), lambda i: (0, i, 0)),
            pl.BlockSpec((1, _BLK, _H), lambda i: (1, i, 0)),
            pl.BlockSpec((_BLK, 1), lambda i: (i, 0)),
        ],
        out_specs=pl.BlockSpec((_G, _H), lambda i: (0, 0)),
        out_shape=jax.ShapeDtypeStruct((_G, _H), jnp.float32),
    )(aggp, aggp, bt)


def _upd_body(h_ref, l_ref, a0_ref, a1_ref, bt_ref, g_ref, wc_ref, bc_ref,
              ws_ref, wd_ref, *out_refs, want_ab):
    hc = a0_ref[0] + a1_ref[0]
    conf = jax.nn.sigmoid(_f32dot(g_ref[...], wc_ref[...]) + bc_ref[...])
    m = (bt_ref[...] == lax.broadcasted_iota(jnp.int32, (_BLK, _G), 1)
         ).astype(jnp.float32)
    cb = _f32dot(m, conf)
    left = l_ref[...]
    hn = h_ref[...] + left * cb * hc
    out_refs[0][...] = hn
    if want_ab:
        out_refs[1][...] = left * (1.0 - cb)
        out_refs[2][...] = _f32dot(hn, ws_ref[...])
        out_refs[3][...] = _f32dot(hn, wd_ref[...])


def _upd(h, left, aggp, bt, g, Wc, bc, Ws, Wd, want_ab):
    n_out = 4 if want_ab else 1
    out_shapes = [jax.ShapeDtypeStruct((_N, _H), jnp.float32),
                  jax.ShapeDtypeStruct((_N, 1), jnp.float32),
                  jax.ShapeDtypeStruct((_N, _H), jnp.float32),
                  jax.ShapeDtypeStruct((_N, _H), jnp.float32)][:n_out]
    out_specs = [pl.BlockSpec((_BLK, _H), lambda i: (i, 0)),
                 pl.BlockSpec((_BLK, 1), lambda i: (i, 0)),
                 pl.BlockSpec((_BLK, _H), lambda i: (i, 0)),
                 pl.BlockSpec((_BLK, _H), lambda i: (i, 0))][:n_out]
    return pl.pallas_call(
        functools.partial(_upd_body, want_ab=want_ab),
        grid=(_N // _BLK,),
        in_specs=[
            pl.BlockSpec((_BLK, _H), lambda i: (i, 0)),
            pl.BlockSpec((_BLK, 1), lambda i: (i, 0)),
            pl.BlockSpec((1, _BLK, _H), lambda i: (0, i, 0)),
            pl.BlockSpec((1, _BLK, _H), lambda i: (1, i, 0)),
            pl.BlockSpec((_BLK, 1), lambda i: (i, 0)),
            pl.BlockSpec((_G, _H), lambda i: (0, 0)),
            pl.BlockSpec((_H, 1), lambda i: (0, 0)),
            pl.BlockSpec((1, 1), lambda i: (0, 0)),
            pl.BlockSpec((_D, _H), lambda i: (0, 0)),
            pl.BlockSpec((_D, _H), lambda i: (0, 0)),
        ],
        out_specs=out_specs,
        out_shape=out_shapes,
    )(h, left, aggp, aggp, bt, g, Wc, bc, Ws, Wd)


def _s1_body(sx_ref, h_ref, wah_ref, bt_ref, sc_ref, smax_ref):
    @pl.when(pl.program_id(0) == 0)
    def _():
        smax_ref[...] = jnp.full_like(smax_ref, -1e30)

    s = sx_ref[...] + _f32dot(h_ref[...], wah_ref[...])
    sc_ref[...] = s
    mask = bt_ref[...] == lax.broadcasted_iota(jnp.int32, (_BLK, _G), 1)
    v = jnp.where(mask, s, -1e30)
    bm = jnp.max(v, axis=0)
    smax_ref[...] = jnp.maximum(smax_ref[...],
                                jnp.broadcast_to(bm[:, None], (_G, _H)))


def _s1(sx, h, Wah, bt):
    return pl.pallas_call(
        _s1_body,
        grid=(_N // _BLK,),
        in_specs=[
            pl.BlockSpec((_BLK, 1), lambda i: (i, 0)),
            pl.BlockSpec((_BLK, _H), lambda i: (i, 0)),
            pl.BlockSpec((_H, 1), lambda i: (0, 0)),
            pl.BlockSpec((_BLK, 1), lambda i: (i, 0)),
        ],
        out_specs=[
            pl.BlockSpec((_BLK, 1), lambda i: (i, 0)),
            pl.BlockSpec((_G, _H), lambda i: (0, 0)),
        ],
        out_shape=[
            jax.ShapeDtypeStruct((_N, 1), jnp.float32),
            jax.ShapeDtypeStruct((_G, _H), jnp.float32),
        ],
    )(sx, h, Wah, bt)


def _s2_body(sc_ref, h_ref, bt_ref, smax_ref, gf_ref, den_ref, cnt_ref):
    @pl.when(pl.program_id(0) == 0)
    def _():
        gf_ref[...] = jnp.zeros_like(gf_ref)
        den_ref[...] = jnp.zeros_like(den_ref)
        cnt_ref[...] = jnp.zeros_like(cnt_ref)

    mf = (bt_ref[...] == lax.broadcasted_iota(jnp.int32, (_BLK, _G), 1)
          ).astype(jnp.float32)
    smax_col = smax_ref[...][:, 0:1]
    smax_row = _f32dot(mf, smax_col)
    e = jnp.exp(sc_ref[...] - smax_row)
    w = mf * e
    den_ref[...] += jnp.broadcast_to(jnp.sum(w, axis=0)[:, None], (_G, _H))
    cnt_ref[...] += jnp.broadcast_to(jnp.sum(mf, axis=0)[:, None], (_G, _H))
    gf_ref[...] += lax.dot_general(w, h_ref[...], (((0,), (0,)), ((), ())),
                                   preferred_element_type=jnp.float32)


def _s2(scores, h, bt, smax):
    return pl.pallas_call(
        _s2_body,
        grid=(_N // _BLK,),
        in_specs=[
            pl.BlockSpec((_BLK, 1), lambda i: (i, 0)),
            pl.BlockSpec((_BLK, _H), lambda i: (i, 0)),
            pl.BlockSpec((_BLK, 1), lambda i: (i, 0)),
            pl.BlockSpec((_G, _H), lambda i: (0, 0)),
        ],
        out_specs=[
            pl.BlockSpec((_G, _H), lambda i: (0, 0)),
            pl.BlockSpec((_G, _H), lambda i: (0, 0)),
            pl.BlockSpec((_G, _H), lambda i: (0, 0)),
        ],
        out_shape=[
            jax.ShapeDtypeStruct((_G, _H), jnp.float32),
            jax.ShapeDtypeStruct((_G, _H), jnp.float32),
            jax.ShapeDtypeStruct((_G, _H), jnp.float32),
        ],
    )(scores, h, bt, smax)


def _s3_body(gf_ref, den_ref, cnt_ref, wh_ref, bh_ref, out_ref, cnt_out_ref):
    den = den_ref[...][:, 0:1] + 1e-16
    gfeat = gf_ref[...] / den
    out_ref[...] = _f32dot(gfeat, wh_ref[...]) + bh_ref[...]
    cnt_out_ref[...] = cnt_ref[...][:, 0:1]


def _s3(gf, den, cnt, Wh, bh):
    return pl.pallas_call(
        _s3_body,
        grid=(1,),
        in_specs=[
            pl.BlockSpec((_G, _H), lambda i: (0, 0)),
            pl.BlockSpec((_G, _H), lambda i: (0, 0)),
            pl.BlockSpec((_G, _H), lambda i: (0, 0)),
            pl.BlockSpec((_H, 1), lambda i: (0, 0)),
            pl.BlockSpec((1, 1), lambda i: (0, 0)),
        ],
        out_specs=[
            pl.BlockSpec((_G, 1), lambda i: (0, 0)),
            pl.BlockSpec((_G, 1), lambda i: (0, 0)),
        ],
        out_shape=[
            jax.ShapeDtypeStruct((_G, 1), jnp.float32),
            jax.ShapeDtypeStruct((_G, 1), jnp.float32),
        ],
    )(gf, den, cnt, Wh, bh)


def kernel(x, edge_index, edge_attr, batch, W_emb, b_emb, W_msg, b_msg,
           W_conf, b_conf, W_att, b_att, W_head, b_head):
    src = edge_index[0]
    dst = edge_index[1]
    Ws = W_msg[:_H]
    Wd = W_msg[_H:2 * _H]
    We = W_msg[2 * _H:]
    Wax = W_att[:_D]
    Wah = W_att[_D:]
    bt = batch.reshape(_N, 1)
    be = b_emb.reshape(1, _H)
    bm = b_msg.reshape(1, _H)
    bc = b_conf.reshape(1, 1)
    ba = b_att.reshape(1, 1)
    bh = b_head.reshape(1, 1)

    h, A, B, sx = _pre(x, W_emb, be, Ws, Wd, Wax, ba)
    C = _edgec(edge_attr, We, bm)
    left = jnp.ones((_N, 1), jnp.float32)

    for it in range(_ITERS):
        aggp = _sc_edge(A, B, C, src, dst)
        g = _gred(aggp, bt)
        if it < _ITERS - 1:
            h, left, A, B = _upd(h, left, aggp, bt, g, W_conf, bc, Ws, Wd,
                                 want_ab=True)
        else:
            (h,) = _upd(h, left, aggp, bt, g, W_conf, bc, Ws, Wd,
                        want_ab=False)

    scores, smax = _s1(sx, h, Wah, bt)
    gf, den, cnt = _s2(scores, h, bt, smax)
    out, counts = _s3(gf, den, cnt, W_head, bh)
    return out, counts


# speedup vs baseline: 1.3076x; 1.3076x over previous
import functools

import numpy as np

import jax
import jax.numpy as jnp
from jax import lax
from jax.experimental import pallas as pl
from jax.experimental.pallas import tpu as pltpu
from jax.experimental.pallas import tpu_sc as plsc

_N = 10000
_E = 320000
_D = 128
_DE = 16
_H = 128
_G = 16
_ITERS = 5

_BLK = 2000
_EBLK = 4000
_K = 40
_NTILES = 32
_EPT = _E // _NTILES
_NCH = _EPT // _K
_SCN = 10240
_RPT = _SCN // 16


def _f32dot(a, b):
    return jnp.dot(a, b, preferred_element_type=jnp.float32)


def _pre_body(x_ref, we_ref, be_ref, ws_ref, wd_ref, wax_ref, ba_ref,
              h0_ref, a_ref, b_ref, sx_ref):
    x = x_ref[...]
    h0 = _f32dot(x, we_ref[...]) + be_ref[...]
    h0_ref[...] = h0
    a_ref[...] = _f32dot(h0, ws_ref[...])
    b_ref[...] = _f32dot(h0, wd_ref[...])
    sx_ref[...] = _f32dot(x, wax_ref[...]) + ba_ref[...]


def _pre(x, W_emb, be, Ws, Wd, Wax, ba):
    n_blocks = _N // _BLK
    return pl.pallas_call(
        _pre_body,
        grid=(n_blocks,),
        in_specs=[
            pl.BlockSpec((_BLK, _D), lambda i: (i, 0)),
            pl.BlockSpec((_D, _H), lambda i: (0, 0)),
            pl.BlockSpec((1, _H), lambda i: (0, 0)),
            pl.BlockSpec((_D, _H), lambda i: (0, 0)),
            pl.BlockSpec((_D, _H), lambda i: (0, 0)),
            pl.BlockSpec((_D, 1), lambda i: (0, 0)),
            pl.BlockSpec((1, 1), lambda i: (0, 0)),
        ],
        out_specs=[
            pl.BlockSpec((_BLK, _H), lambda i: (i, 0)),
            pl.BlockSpec((_BLK, _H), lambda i: (i, 0)),
            pl.BlockSpec((_BLK, _H), lambda i: (i, 0)),
            pl.BlockSpec((_BLK, 1), lambda i: (i, 0)),
        ],
        out_shape=[
            jax.ShapeDtypeStruct((_N, _H), jnp.float32),
            jax.ShapeDtypeStruct((_N, _H), jnp.float32),
            jax.ShapeDtypeStruct((_N, _H), jnp.float32),
            jax.ShapeDtypeStruct((_N, 1), jnp.float32),
        ],
    )(x, W_emb, be, Ws, Wd, Wax, ba)


def _edgec_body(ea_ref, we_ref, bm_ref, c_ref):
    c_ref[...] = _f32dot(ea_ref[...], we_ref[...]) + bm_ref[...]


def _edgec(edge_attr, We, bm):
    return pl.pallas_call(
        _edgec_body,
        grid=(_E // _EBLK,),
        in_specs=[
            pl.BlockSpec((_EBLK, _DE), lambda i: (i, 0)),
            pl.BlockSpec((_DE, _H), lambda i: (0, 0)),
            pl.BlockSpec((1, _H), lambda i: (0, 0)),
        ],
        out_specs=pl.BlockSpec((_EBLK, _H), lambda i: (i, 0)),
        out_shape=jax.ShapeDtypeStruct((_E, _H), jnp.float32),
    )(edge_attr, We, bm)


def _sc_edge_body(a_hbm, b_hbm, c_hbm, src_hbm, dst_hbm, out_hbm,
                  idxs0, idxs1, idxd0, idxd1, bufa0, bufa1, bufb0, bufb1,
                  bufc0, bufc1, shared,
                  sema0, sema1, semb0, semb1, semc0, semc1,
                  semis0, semis1, semid0, semid1):
    c = lax.axis_index("c")
    s = lax.axis_index("s")
    wid = c * 16 + s
    idxs = (idxs0, idxs1)
    idxd = (idxd0, idxd1)
    bufa = (bufa0, bufa1)
    bufb = (bufb0, bufb1)
    bufc = (bufc0, bufc1)
    sema = (sema0, sema1)
    semb = (semb0, semb1)
    semc = (semc0, semc1)
    semis = (semis0, semis1)
    semid = (semid0, semid1)

    zero16 = jnp.zeros((16,), jnp.float32)

    def zrow(r, carry):
        for jj in range(8):
            bufa0[r, pl.ds(jj * 16, 16)] = zero16
        return carry

    lax.fori_loop(0, _K, zrow, 0)
    row0 = s * _RPT
    for t in range(_RPT // _K):
        pltpu.sync_copy(bufa0, shared.at[pl.ds(row0 + t * _K, _K)])
    plsc.subcore_barrier()

    ebase = wid * _EPT

    def idx_issue(p, j):
        pltpu.async_copy(src_hbm.at[pl.ds(ebase + j * _K, _K)], idxs[p],
                         semis[p])
        pltpu.async_copy(dst_hbm.at[pl.ds(ebase + j * _K, _K)], idxd[p],
                         semid[p])

    def idx_wait(p, j):
        pltpu.make_async_copy(src_hbm.at[pl.ds(ebase + j * _K, _K)], idxs[p],
                              semis[p]).wait()
        pltpu.make_async_copy(dst_hbm.at[pl.ds(ebase + j * _K, _K)], idxd[p],
                              semid[p]).wait()

    def gather_issue(p, j):
        pltpu.async_copy(a_hbm.at[idxs[p]], bufa[p], sema[p])
        pltpu.async_copy(b_hbm.at[idxd[p]], bufb[p], semb[p])
        pltpu.async_copy(c_hbm.at[pl.ds(ebase + j * _K, _K)], bufc[p],
                         semc[p])

    def gather_wait(p, j):
        pltpu.make_async_copy(a_hbm.at[idxs[p]], bufa[p], sema[p]).wait()
        pltpu.make_async_copy(b_hbm.at[idxd[p]], bufb[p], semb[p]).wait()
        pltpu.make_async_copy(c_hbm.at[pl.ds(ebase + j * _K, _K)], bufc[p],
                              semc[p]).wait()

    def compute(p):
        a, b, m = bufa[p], bufb[p], bufc[p]

        def row(r, rc):
            for jj in range(8):
                sl = pl.ds(jj * 16, 16)
                m[r, sl] = jnp.maximum(a[r, sl] + b[r, sl] + m[r, sl], 0.0)
            return rc

        lax.fori_loop(0, _K, row, 0)

    def visit(p, j):
        q = 1 - p
        gather_wait(p, j)

        @pl.when(j + 1 < _NCH)
        def _():
            idx_wait(q, j + 1)
            gather_issue(q, j + 1)

        compute(p)
        pltpu.sync_copy(bufc[p], shared.at[idxd[p]], add=True)

        @pl.when(j + 2 < _NCH)
        def _():
            idx_issue(p, j + 2)

    idx_issue(0, 0)
    idx_issue(1, 1)
    idx_wait(0, 0)
    gather_issue(0, 0)

    def pair(t, carry):
        j = 2 * t
        visit(0, j)
        visit(1, j + 1)
        return carry

    lax.fori_loop(0, _NCH // 2, pair, 0)
    if _NCH % 2:
        visit(0, _NCH - 1)
    plsc.subcore_barrier()

    for t in range(_RPT // _K):
        pltpu.sync_copy(shared.at[pl.ds(row0 + t * _K, _K)],
                        out_hbm.at[c, pl.ds(row0 + t * _K, _K)])


def _sc_edge(A, B, C, src, dst):
    mesh = plsc.VectorSubcoreMesh(core_axis_name="c", subcore_axis_name="s")
    fn = functools.partial(
        pl.kernel,
        out_type=jax.ShapeDtypeStruct((2, _SCN, _H), jnp.float32),
        mesh=mesh,
        scratch_types=[
            pltpu.VMEM((_K,), jnp.int32),
            pltpu.VMEM((_K,), jnp.int32),
            pltpu.VMEM((_K,), jnp.int32),
            pltpu.VMEM((_K,), jnp.int32),
            pltpu.VMEM((_K, _H), jnp.float32),
            pltpu.VMEM((_K, _H), jnp.float32),
            pltpu.VMEM((_K, _H), jnp.float32),
            pltpu.VMEM((_K, _H), jnp.float32),
            pltpu.VMEM((_K, _H), jnp.float32),
            pltpu.VMEM((_K, _H), jnp.float32),
            pltpu.VMEM_SHARED((_SCN, _H), jnp.float32),
        ] + [pltpu.SemaphoreType.DMA] * 10,
    )(_sc_edge_body)
    return fn(A, B, C, src, dst)


def _gred_body(a0_ref, a1_ref, bt_ref, g_ref):
    @pl.when(pl.program_id(0) == 0)
    def _():
        g_ref[...] = jnp.zeros_like(g_ref)

    hc = a0_ref[0] + a1_ref[0]
    m = (bt_ref[...] == lax.broadcasted_iota(jnp.int32, (_BLK, _G), 1)
         ).astype(jnp.float32)
    g_ref[...] += lax.dot_general(m, hc, (((0,), (0,)), ((), ())),
                                  preferred_element_type=jnp.float32)


def _gred(aggp, bt):
    return pl.pallas_call(
        _gred_body,
        grid=(_N // _BLK,),
        in_specs=[
            pl.BlockSpec((1, _BLK, _H), lambda i: (0, i, 0)),
            pl.BlockSpec((1, _BLK, _H), lambda i: (1, i, 0)),
            pl.BlockSpec((_BLK, 1), lambda i: (i, 0)),
        ],
        out_specs=pl.BlockSpec((_G, _H), lambda i: (0, 0)),
        out_shape=jax.ShapeDtypeStruct((_G, _H), jnp.float32),
    )(aggp, aggp, bt)


def _upd_body(h_ref, l_ref, a0_ref, a1_ref, bt_ref, g_ref, wc_ref, bc_ref,
              ws_ref, wd_ref, *out_refs, want_ab):
    hc = a0_ref[0] + a1_ref[0]
    conf = jax.nn.sigmoid(_f32dot(g_ref[...], wc_ref[...]) + bc_ref[...])
    m = (bt_ref[...] == lax.broadcasted_iota(jnp.int32, (_BLK, _G), 1)
         ).astype(jnp.float32)
    cb = _f32dot(m, conf)
    left = l_ref[...]
    hn = h_ref[...] + left * cb * hc
    out_refs[0][...] = hn
    if want_ab:
        out_refs[1][...] = left * (1.0 - cb)
        out_refs[2][...] = _f32dot(hn, ws_ref[...])
        out_refs[3][...] = _f32dot(hn, wd_ref[...])


def _upd(h, left, aggp, bt, g, Wc, bc, Ws, Wd, want_ab):
    n_out = 4 if want_ab else 1
    out_shapes = [jax.ShapeDtypeStruct((_N, _H), jnp.float32),
                  jax.ShapeDtypeStruct((_N, 1), jnp.float32),
                  jax.ShapeDtypeStruct((_N, _H), jnp.float32),
                  jax.ShapeDtypeStruct((_N, _H), jnp.float32)][:n_out]
    out_specs = [pl.BlockSpec((_BLK, _H), lambda i: (i, 0)),
                 pl.BlockSpec((_BLK, 1), lambda i: (i, 0)),
                 pl.BlockSpec((_BLK, _H), lambda i: (i, 0)),
                 pl.BlockSpec((_BLK, _H), lambda i: (i, 0))][:n_out]
    return pl.pallas_call(
        functools.partial(_upd_body, want_ab=want_ab),
        grid=(_N // _BLK,),
        in_specs=[
            pl.BlockSpec((_BLK, _H), lambda i: (i, 0)),
            pl.BlockSpec((_BLK, 1), lambda i: (i, 0)),
            pl.BlockSpec((1, _BLK, _H), lambda i: (0, i, 0)),
            pl.BlockSpec((1, _BLK, _H), lambda i: (1, i, 0)),
            pl.BlockSpec((_BLK, 1), lambda i: (i, 0)),
            pl.BlockSpec((_G, _H), lambda i: (0, 0)),
            pl.BlockSpec((_H, 1), lambda i: (0, 0)),
            pl.BlockSpec((1, 1), lambda i: (0, 0)),
            pl.BlockSpec((_D, _H), lambda i: (0, 0)),
            pl.BlockSpec((_D, _H), lambda i: (0, 0)),
        ],
        out_specs=out_specs,
        out_shape=out_shapes,
    )(h, left, aggp, aggp, bt, g, Wc, bc, Ws, Wd)


def _updf_body(h_ref, l_ref, a0_ref, a1_ref, bt_ref, g_ref, wc_ref, bc_ref,
               sx_ref, wah_ref, hn_ref, sc_ref, smax_ref):
    @pl.when(pl.program_id(0) == 0)
    def _():
        smax_ref[...] = jnp.full_like(smax_ref, -1e30)

    hc = a0_ref[0] + a1_ref[0]
    conf = jax.nn.sigmoid(_f32dot(g_ref[...], wc_ref[...]) + bc_ref[...])
    mask = bt_ref[...] == lax.broadcasted_iota(jnp.int32, (_BLK, _G), 1)
    cb = _f32dot(mask.astype(jnp.float32), conf)
    hn = h_ref[...] + l_ref[...] * cb * hc
    hn_ref[...] = hn
    s = sx_ref[...] + _f32dot(hn, wah_ref[...])
    sc_ref[...] = s
    v = jnp.where(mask, s, -1e30)
    bm = jnp.max(v, axis=0)
    smax_ref[...] = jnp.maximum(smax_ref[...],
                                jnp.broadcast_to(bm[:, None], (_G, _H)))


def _updf(h, left, aggp, bt, g, Wc, bc, sx, Wah):
    return pl.pallas_call(
        _updf_body,
        grid=(_N // _BLK,),
        in_specs=[
            pl.BlockSpec((_BLK, _H), lambda i: (i, 0)),
            pl.BlockSpec((_BLK, 1), lambda i: (i, 0)),
            pl.BlockSpec((1, _BLK, _H), lambda i: (0, i, 0)),
            pl.BlockSpec((1, _BLK, _H), lambda i: (1, i, 0)),
            pl.BlockSpec((_BLK, 1), lambda i: (i, 0)),
            pl.BlockSpec((_G, _H), lambda i: (0, 0)),
            pl.BlockSpec((_H, 1), lambda i: (0, 0)),
            pl.BlockSpec((1, 1), lambda i: (0, 0)),
            pl.BlockSpec((_BLK, 1), lambda i: (i, 0)),
            pl.BlockSpec((_H, 1), lambda i: (0, 0)),
        ],
        out_specs=[
            pl.BlockSpec((_BLK, _H), lambda i: (i, 0)),
            pl.BlockSpec((_BLK, 1), lambda i: (i, 0)),
            pl.BlockSpec((_G, _H), lambda i: (0, 0)),
        ],
        out_shape=[
            jax.ShapeDtypeStruct((_N, _H), jnp.float32),
            jax.ShapeDtypeStruct((_N, 1), jnp.float32),
            jax.ShapeDtypeStruct((_G, _H), jnp.float32),
        ],
    )(h, left, aggp, aggp, bt, g, Wc, bc, sx, Wah)


def _s2_body(sc_ref, h_ref, bt_ref, smax_ref, wh_ref, bh_ref,
             out_ref, cnt_out_ref, gf_ref, den_ref, cnt_ref):
    @pl.when(pl.program_id(0) == 0)
    def _():
        gf_ref[...] = jnp.zeros_like(gf_ref)
        den_ref[...] = jnp.zeros_like(den_ref)
        cnt_ref[...] = jnp.zeros_like(cnt_ref)

    mf = (bt_ref[...] == lax.broadcasted_iota(jnp.int32, (_BLK, _G), 1)
          ).astype(jnp.float32)
    smax_col = smax_ref[...][:, 0:1]
    smax_row = _f32dot(mf, smax_col)
    e = jnp.exp(sc_ref[...] - smax_row)
    w = mf * e
    den_ref[...] += jnp.broadcast_to(jnp.sum(w, axis=0)[:, None], (_G, _H))
    cnt_ref[...] += jnp.broadcast_to(jnp.sum(mf, axis=0)[:, None], (_G, _H))
    gf_ref[...] += lax.dot_general(w, h_ref[...], (((0,), (0,)), ((), ())),
                                   preferred_element_type=jnp.float32)

    @pl.when(pl.program_id(0) == _N // _BLK - 1)
    def _():
        den = den_ref[...][:, 0:1] + 1e-16
        gfeat = gf_ref[...] / den
        out_ref[...] = _f32dot(gfeat, wh_ref[...]) + bh_ref[...]
        cnt_out_ref[...] = cnt_ref[...][:, 0:1]


def _s2(scores, h, bt, smax, Wh, bh):
    return pl.pallas_call(
        _s2_body,
        grid=(_N // _BLK,),
        in_specs=[
            pl.BlockSpec((_BLK, 1), lambda i: (i, 0)),
            pl.BlockSpec((_BLK, _H), lambda i: (i, 0)),
            pl.BlockSpec((_BLK, 1), lambda i: (i, 0)),
            pl.BlockSpec((_G, _H), lambda i: (0, 0)),
            pl.BlockSpec((_H, 1), lambda i: (0, 0)),
            pl.BlockSpec((1, 1), lambda i: (0, 0)),
        ],
        out_specs=[
            pl.BlockSpec((_G, 1), lambda i: (0, 0)),
            pl.BlockSpec((_G, 1), lambda i: (0, 0)),
        ],
        out_shape=[
            jax.ShapeDtypeStruct((_G, 1), jnp.float32),
            jax.ShapeDtypeStruct((_G, 1), jnp.float32),
        ],
        scratch_shapes=[
            pltpu.VMEM((_G, _H), jnp.float32),
            pltpu.VMEM((_G, _H), jnp.float32),
            pltpu.VMEM((_G, _H), jnp.float32),
        ],
    )(scores, h, bt, smax, Wh, bh)


def kernel(x, edge_index, edge_attr, batch, W_emb, b_emb, W_msg, b_msg,
           W_conf, b_conf, W_att, b_att, W_head, b_head):
    src = edge_index[0]
    dst = edge_index[1]
    Ws = W_msg[:_H]
    Wd = W_msg[_H:2 * _H]
    We = W_msg[2 * _H:]
    Wax = W_att[:_D]
    Wah = W_att[_D:]
    bt = batch.reshape(_N, 1)
    be = b_emb.reshape(1, _H)
    bm = b_msg.reshape(1, _H)
    bc = b_conf.reshape(1, 1)
    ba = b_att.reshape(1, 1)
    bh = b_head.reshape(1, 1)

    h, A, B, sx = _pre(x, W_emb, be, Ws, Wd, Wax, ba)
    C = _edgec(edge_attr, We, bm)
    left = jnp.ones((_N, 1), jnp.float32)

    for it in range(_ITERS):
        aggp = _sc_edge(A, B, C, src, dst)
        g = _gred(aggp, bt)
        if it < _ITERS - 1:
            h, left, A, B = _upd(h, left, aggp, bt, g, W_conf, bc, Ws, Wd,
                                 want_ab=True)
        else:
            h, scores, smax = _updf(h, left, aggp, bt, g, W_conf, bc,
                                    sx, Wah)

    out, counts = _s2(scores, h, bt, smax, W_head, bh)
    return out, counts


# speedup vs baseline: 1.3444x; 1.0282x over previous
import functools

import numpy as np

import jax
import jax.numpy as jnp
from jax import lax
from jax.experimental import pallas as pl
from jax.experimental.pallas import tpu as pltpu
from jax.experimental.pallas import tpu_sc as plsc

_N = 10000
_E = 320000
_D = 128
_DE = 16
_H = 128
_G = 16
_ITERS = 5

_BLK = 2000
_EBLK = 4000
_K = 40
_NTILES = 32
_EPT = _E // _NTILES
_NCH = _EPT // _K
_SCN = 10240
_RPT = _SCN // 16


def _f32dot(a, b):
    return jnp.dot(a, b, preferred_element_type=jnp.float32)


def _pre_body(x_ref, we_ref, be_ref, ws_ref, wd_ref, wax_ref, ba_ref,
              h0_ref, a_ref, b_ref, sx_ref):
    x = x_ref[...]
    h0 = _f32dot(x, we_ref[...]) + be_ref[...]
    h0_ref[...] = h0
    a_ref[...] = _f32dot(h0, ws_ref[...])
    b_ref[...] = _f32dot(h0, wd_ref[...])
    sx_ref[...] = _f32dot(x, wax_ref[...]) + ba_ref[...]


def _pre(x, W_emb, be, Ws, Wd, Wax, ba):
    n_blocks = _N // _BLK
    return pl.pallas_call(
        _pre_body,
        grid=(n_blocks,),
        in_specs=[
            pl.BlockSpec((_BLK, _D), lambda i: (i, 0)),
            pl.BlockSpec((_D, _H), lambda i: (0, 0)),
            pl.BlockSpec((1, _H), lambda i: (0, 0)),
            pl.BlockSpec((_D, _H), lambda i: (0, 0)),
            pl.BlockSpec((_D, _H), lambda i: (0, 0)),
            pl.BlockSpec((_D, 1), lambda i: (0, 0)),
            pl.BlockSpec((1, 1), lambda i: (0, 0)),
        ],
        out_specs=[
            pl.BlockSpec((_BLK, _H), lambda i: (i, 0)),
            pl.BlockSpec((_BLK, _H), lambda i: (i, 0)),
            pl.BlockSpec((_BLK, _H), lambda i: (i, 0)),
            pl.BlockSpec((_BLK, 1), lambda i: (i, 0)),
        ],
        out_shape=[
            jax.ShapeDtypeStruct((_N, _H), jnp.float32),
            jax.ShapeDtypeStruct((_N, _H), jnp.float32),
            jax.ShapeDtypeStruct((_N, _H), jnp.float32),
            jax.ShapeDtypeStruct((_N, 1), jnp.float32),
        ],
    )(x, W_emb, be, Ws, Wd, Wax, ba)


def _edgec_body(ea_ref, we_ref, bm_ref, c_ref):
    c_ref[...] = _f32dot(ea_ref[...], we_ref[...]) + bm_ref[...]


def _edgec(edge_attr, We, bm):
    return pl.pallas_call(
        _edgec_body,
        grid=(_E // _EBLK,),
        in_specs=[
            pl.BlockSpec((_EBLK, _DE), lambda i: (i, 0)),
            pl.BlockSpec((_DE, _H), lambda i: (0, 0)),
            pl.BlockSpec((1, _H), lambda i: (0, 0)),
        ],
        out_specs=pl.BlockSpec((_EBLK, _H), lambda i: (i, 0)),
        out_shape=jax.ShapeDtypeStruct((_E, _H), jnp.float32),
    )(edge_attr, We, bm)


def _sc_edge_body(a_hbm, b_hbm, c_hbm, src_hbm, dst_hbm, out_hbm,
                  idxs0, idxs1, idxs2, idxd0, idxd1, idxd2,
                  bufa0, bufa1, bufa2, bufb0, bufb1, bufb2,
                  bufc0, bufc1, bufc2, shared,
                  sema0, sema1, sema2, semb0, semb1, semb2,
                  semc0, semc1, semc2, semis0, semis1, semis2,
                  semid0, semid1, semid2):
    c = lax.axis_index("c")
    s = lax.axis_index("s")
    wid = c * 16 + s
    idxs = (idxs0, idxs1, idxs2)
    idxd = (idxd0, idxd1, idxd2)
    bufa = (bufa0, bufa1, bufa2)
    bufb = (bufb0, bufb1, bufb2)
    bufc = (bufc0, bufc1, bufc2)
    sema = (sema0, sema1, sema2)
    semb = (semb0, semb1, semb2)
    semc = (semc0, semc1, semc2)
    semis = (semis0, semis1, semis2)
    semid = (semid0, semid1, semid2)

    zero16 = jnp.zeros((16,), jnp.float32)

    def zrow(r, carry):
        for jj in range(8):
            bufa0[r, pl.ds(jj * 16, 16)] = zero16
        return carry

    lax.fori_loop(0, _K, zrow, 0)
    row0 = s * _RPT
    for t in range(_RPT // _K):
        pltpu.sync_copy(bufa0, shared.at[pl.ds(row0 + t * _K, _K)])
    plsc.subcore_barrier()

    ebase = wid * _EPT

    def idx_issue(p, j):
        pltpu.async_copy(src_hbm.at[pl.ds(ebase + j * _K, _K)], idxs[p],
                         semis[p])
        pltpu.async_copy(dst_hbm.at[pl.ds(ebase + j * _K, _K)], idxd[p],
                         semid[p])

    def idx_wait(p, j):
        pltpu.make_async_copy(src_hbm.at[pl.ds(ebase + j * _K, _K)], idxs[p],
                              semis[p]).wait()
        pltpu.make_async_copy(dst_hbm.at[pl.ds(ebase + j * _K, _K)], idxd[p],
                              semid[p]).wait()

    def gather_issue(p, j):
        pltpu.async_copy(a_hbm.at[idxs[p]], bufa[p], sema[p])
        pltpu.async_copy(b_hbm.at[idxd[p]], bufb[p], semb[p])
        pltpu.async_copy(c_hbm.at[pl.ds(ebase + j * _K, _K)], bufc[p],
                         semc[p])

    def gather_wait(p, j):
        pltpu.make_async_copy(a_hbm.at[idxs[p]], bufa[p], sema[p]).wait()
        pltpu.make_async_copy(b_hbm.at[idxd[p]], bufb[p], semb[p]).wait()
        pltpu.make_async_copy(c_hbm.at[pl.ds(ebase + j * _K, _K)], bufc[p],
                              semc[p]).wait()

    def compute(p):
        a, b, m = bufa[p], bufb[p], bufc[p]

        def row(r, rc):
            for jj in range(8):
                sl = pl.ds(jj * 16, 16)
                m[r, sl] = jnp.maximum(a[r, sl] + b[r, sl] + m[r, sl], 0.0)
            return rc

        lax.fori_loop(0, _K, row, 0)

    def visit(p, j):
        q = (p + 2) % 3
        gather_wait(p, j)

        @pl.when(j + 2 < _NCH)
        def _():
            idx_wait(q, j + 2)
            gather_issue(q, j + 2)

        compute(p)
        pltpu.sync_copy(bufc[p], shared.at[idxd[p]], add=True)

        @pl.when(j + 3 < _NCH)
        def _():
            idx_issue(p, j + 3)

    idx_issue(0, 0)
    idx_issue(1, 1)
    idx_issue(2, 2)
    idx_wait(0, 0)
    gather_issue(0, 0)
    idx_wait(1, 1)
    gather_issue(1, 1)

    def triple(t, carry):
        j = 3 * t
        visit(0, j)
        visit(1, j + 1)
        visit(2, j + 2)
        return carry

    lax.fori_loop(0, _NCH // 3, triple, 0)
    for jtail in range((_NCH // 3) * 3, _NCH):
        visit(jtail % 3, jtail)
    plsc.subcore_barrier()

    for t in range(_RPT // _K):
        pltpu.sync_copy(shared.at[pl.ds(row0 + t * _K, _K)],
                        out_hbm.at[c, pl.ds(row0 + t * _K, _K)])


def _sc_edge(A, B, C, src, dst):
    mesh = plsc.VectorSubcoreMesh(core_axis_name="c", subcore_axis_name="s")
    fn = functools.partial(
        pl.kernel,
        out_type=jax.ShapeDtypeStruct((2, _SCN, _H), jnp.float32),
        mesh=mesh,
        scratch_types=(
            [pltpu.VMEM((_K,), jnp.int32)] * 6
            + [pltpu.VMEM((_K, _H), jnp.float32)] * 9
            + [pltpu.VMEM_SHARED((_SCN, _H), jnp.float32)]
            + [pltpu.SemaphoreType.DMA] * 15),
    )(_sc_edge_body)
    return fn(A, B, C, src, dst)


def _gred_body(a0_ref, a1_ref, bt_ref, g_ref):
    @pl.when(pl.program_id(0) == 0)
    def _():
        g_ref[...] = jnp.zeros_like(g_ref)

    hc = a0_ref[0] + a1_ref[0]
    m = (bt_ref[...] == lax.broadcasted_iota(jnp.int32, (_BLK, _G), 1)
         ).astype(jnp.float32)
    g_ref[...] += lax.dot_general(m, hc, (((0,), (0,)), ((), ())),
                                  preferred_element_type=jnp.float32)


def _gred(aggp, bt):
    return pl.pallas_call(
        _gred_body,
        grid=(_N // _BLK,),
        in_specs=[
            pl.BlockSpec((1, _BLK, _H), lambda i: (0, i, 0)),
            pl.BlockSpec((1, _BLK, _H), lambda i: (1, i, 0)),
            pl.BlockSpec((_BLK, 1), lambda i: (i, 0)),
        ],
        out_specs=pl.BlockSpec((_G, _H), lambda i: (0, 0)),
        out_shape=jax.ShapeDtypeStruct((_G, _H), jnp.float32),
    )(aggp, aggp, bt)


def _upd_body(h_ref, l_ref, a0_ref, a1_ref, bt_ref, g_ref, wc_ref, bc_ref,
              ws_ref, wd_ref, *out_refs, want_ab):
    hc = a0_ref[0] + a1_ref[0]
    conf = jax.nn.sigmoid(_f32dot(g_ref[...], wc_ref[...]) + bc_ref[...])
    m = (bt_ref[...] == lax.broadcasted_iota(jnp.int32, (_BLK, _G), 1)
         ).astype(jnp.float32)
    cb = _f32dot(m, conf)
    left = l_ref[...]
    hn = h_ref[...] + left * cb * hc
    out_refs[0][...] = hn
    if want_ab:
        out_refs[1][...] = left * (1.0 - cb)
        out_refs[2][...] = _f32dot(hn, ws_ref[...])
        out_refs[3][...] = _f32dot(hn, wd_ref[...])


def _upd(h, left, aggp, bt, g, Wc, bc, Ws, Wd, want_ab):
    n_out = 4 if want_ab else 1
    out_shapes = [jax.ShapeDtypeStruct((_N, _H), jnp.float32),
                  jax.ShapeDtypeStruct((_N, 1), jnp.float32),
                  jax.ShapeDtypeStruct((_N, _H), jnp.float32),
                  jax.ShapeDtypeStruct((_N, _H), jnp.float32)][:n_out]
    out_specs = [pl.BlockSpec((_BLK, _H), lambda i: (i, 0)),
                 pl.BlockSpec((_BLK, 1), lambda i: (i, 0)),
                 pl.BlockSpec((_BLK, _H), lambda i: (i, 0)),
                 pl.BlockSpec((_BLK, _H), lambda i: (i, 0))][:n_out]
    return pl.pallas_call(
        functools.partial(_upd_body, want_ab=want_ab),
        grid=(_N // _BLK,),
        in_specs=[
            pl.BlockSpec((_BLK, _H), lambda i: (i, 0)),
            pl.BlockSpec((_BLK, 1), lambda i: (i, 0)),
            pl.BlockSpec((1, _BLK, _H), lambda i: (0, i, 0)),
            pl.BlockSpec((1, _BLK, _H), lambda i: (1, i, 0)),
            pl.BlockSpec((_BLK, 1), lambda i: (i, 0)),
            pl.BlockSpec((_G, _H), lambda i: (0, 0)),
            pl.BlockSpec((_H, 1), lambda i: (0, 0)),
            pl.BlockSpec((1, 1), lambda i: (0, 0)),
            pl.BlockSpec((_D, _H), lambda i: (0, 0)),
            pl.BlockSpec((_D, _H), lambda i: (0, 0)),
        ],
        out_specs=out_specs,
        out_shape=out_shapes,
    )(h, left, aggp, aggp, bt, g, Wc, bc, Ws, Wd)


def _updf_body(h_ref, l_ref, a0_ref, a1_ref, bt_ref, g_ref, wc_ref, bc_ref,
               sx_ref, wah_ref, hn_ref, sc_ref, smax_ref):
    @pl.when(pl.program_id(0) == 0)
    def _():
        smax_ref[...] = jnp.full_like(smax_ref, -1e30)

    hc = a0_ref[0] + a1_ref[0]
    conf = jax.nn.sigmoid(_f32dot(g_ref[...], wc_ref[...]) + bc_ref[...])
    mask = bt_ref[...] == lax.broadcasted_iota(jnp.int32, (_BLK, _G), 1)
    cb = _f32dot(mask.astype(jnp.float32), conf)
    hn = h_ref[...] + l_ref[...] * cb * hc
    hn_ref[...] = hn
    s = sx_ref[...] + _f32dot(hn, wah_ref[...])
    sc_ref[...] = s
    v = jnp.where(mask, s, -1e30)
    bm = jnp.max(v, axis=0)
    smax_ref[...] = jnp.maximum(smax_ref[...],
                                jnp.broadcast_to(bm[:, None], (_G, _H)))


def _updf(h, left, aggp, bt, g, Wc, bc, sx, Wah):
    return pl.pallas_call(
        _updf_body,
        grid=(_N // _BLK,),
        in_specs=[
            pl.BlockSpec((_BLK, _H), lambda i: (i, 0)),
            pl.BlockSpec((_BLK, 1), lambda i: (i, 0)),
            pl.BlockSpec((1, _BLK, _H), lambda i: (0, i, 0)),
            pl.BlockSpec((1, _BLK, _H), lambda i: (1, i, 0)),
            pl.BlockSpec((_BLK, 1), lambda i: (i, 0)),
            pl.BlockSpec((_G, _H), lambda i: (0, 0)),
            pl.BlockSpec((_H, 1), lambda i: (0, 0)),
            pl.BlockSpec((1, 1), lambda i: (0, 0)),
            pl.BlockSpec((_BLK, 1), lambda i: (i, 0)),
            pl.BlockSpec((_H, 1), lambda i: (0, 0)),
        ],
        out_specs=[
            pl.BlockSpec((_BLK, _H), lambda i: (i, 0)),
            pl.BlockSpec((_BLK, 1), lambda i: (i, 0)),
            pl.BlockSpec((_G, _H), lambda i: (0, 0)),
        ],
        out_shape=[
            jax.ShapeDtypeStruct((_N, _H), jnp.float32),
            jax.ShapeDtypeStruct((_N, 1), jnp.float32),
            jax.ShapeDtypeStruct((_G, _H), jnp.float32),
        ],
    )(h, left, aggp, aggp, bt, g, Wc, bc, sx, Wah)


def _s2_body(sc_ref, h_ref, bt_ref, smax_ref, wh_ref, bh_ref,
             out_ref, cnt_out_ref, gf_ref, den_ref, cnt_ref):
    @pl.when(pl.program_id(0) == 0)
    def _():
        gf_ref[...] = jnp.zeros_like(gf_ref)
        den_ref[...] = jnp.zeros_like(den_ref)
        cnt_ref[...] = jnp.zeros_like(cnt_ref)

    mf = (bt_ref[...] == lax.broadcasted_iota(jnp.int32, (_BLK, _G), 1)
          ).astype(jnp.float32)
    smax_col = smax_ref[...][:, 0:1]
    smax_row = _f32dot(mf, smax_col)
    e = jnp.exp(sc_ref[...] - smax_row)
    w = mf * e
    den_ref[...] += jnp.broadcast_to(jnp.sum(w, axis=0)[:, None], (_G, _H))
    cnt_ref[...] += jnp.broadcast_to(jnp.sum(mf, axis=0)[:, None], (_G, _H))
    gf_ref[...] += lax.dot_general(w, h_ref[...], (((0,), (0,)), ((), ())),
                                   preferred_element_type=jnp.float32)

    @pl.when(pl.program_id(0) == _N // _BLK - 1)
    def _():
        den = den_ref[...][:, 0:1] + 1e-16
        gfeat = gf_ref[...] / den
        out_ref[...] = _f32dot(gfeat, wh_ref[...]) + bh_ref[...]
        cnt_out_ref[...] = cnt_ref[...][:, 0:1]


def _s2(scores, h, bt, smax, Wh, bh):
    return pl.pallas_call(
        _s2_body,
        grid=(_N // _BLK,),
        in_specs=[
            pl.BlockSpec((_BLK, 1), lambda i: (i, 0)),
            pl.BlockSpec((_BLK, _H), lambda i: (i, 0)),
            pl.BlockSpec((_BLK, 1), lambda i: (i, 0)),
            pl.BlockSpec((_G, _H), lambda i: (0, 0)),
            pl.BlockSpec((_H, 1), lambda i: (0, 0)),
            pl.BlockSpec((1, 1), lambda i: (0, 0)),
        ],
        out_specs=[
            pl.BlockSpec((_G, 1), lambda i: (0, 0)),
            pl.BlockSpec((_G, 1), lambda i: (0, 0)),
        ],
        out_shape=[
            jax.ShapeDtypeStruct((_G, 1), jnp.float32),
            jax.ShapeDtypeStruct((_G, 1), jnp.float32),
        ],
        scratch_shapes=[
            pltpu.VMEM((_G, _H), jnp.float32),
            pltpu.VMEM((_G, _H), jnp.float32),
            pltpu.VMEM((_G, _H), jnp.float32),
        ],
    )(scores, h, bt, smax, Wh, bh)


def kernel(x, edge_index, edge_attr, batch, W_emb, b_emb, W_msg, b_msg,
           W_conf, b_conf, W_att, b_att, W_head, b_head):
    src = edge_index[0]
    dst = edge_index[1]
    Ws = W_msg[:_H]
    Wd = W_msg[_H:2 * _H]
    We = W_msg[2 * _H:]
    Wax = W_att[:_D]
    Wah = W_att[_D:]
    bt = batch.reshape(_N, 1)
    be = b_emb.reshape(1, _H)
    bm = b_msg.reshape(1, _H)
    bc = b_conf.reshape(1, 1)
    ba = b_att.reshape(1, 1)
    bh = b_head.reshape(1, 1)

    h, A, B, sx = _pre(x, W_emb, be, Ws, Wd, Wax, ba)
    C = _edgec(edge_attr, We, bm)
    left = jnp.ones((_N, 1), jnp.float32)

    for it in range(_ITERS):
        aggp = _sc_edge(A, B, C, src, dst)
        g = _gred(aggp, bt)
        if it < _ITERS - 1:
            h, left, A, B = _upd(h, left, aggp, bt, g, W_conf, bc, Ws, Wd,
                                 want_ab=True)
        else:
            h, scores, smax = _updf(h, left, aggp, bt, g, W_conf, bc,
                                    sx, Wah)

    out, counts = _s2(scores, h, bt, smax, W_head, bh)
    return out, counts


# speedup vs baseline: 1.6065x; 1.1950x over previous
import functools

import numpy as np

import jax
import jax.numpy as jnp
from jax import lax
from jax.experimental import pallas as pl
from jax.experimental.pallas import tpu as pltpu
from jax.experimental.pallas import tpu_sc as plsc

_N = 10000
_E = 320000
_D = 128
_DE = 16
_H = 128
_G = 16
_ITERS = 5

_BLK = 2000
_EBLK = 4000
_K = 40
_NTILES = 32
_EPT = _E // _NTILES
_NCH = _EPT // _K
_SCN = 10240
_RPT = _SCN // 16


def _f32dot(a, b):
    return jnp.dot(a, b, preferred_element_type=jnp.float32)


def _pre_body(x_ref, we_ref, be_ref, ws_ref, wd_ref, wax_ref, ba_ref,
              h0_ref, a_ref, b_ref, sx_ref):
    x = x_ref[...]
    h0 = _f32dot(x, we_ref[...]) + be_ref[...]
    h0_ref[...] = h0
    a_ref[...] = _f32dot(h0, ws_ref[...])
    b_ref[...] = _f32dot(h0, wd_ref[...])
    sx_ref[...] = _f32dot(x, wax_ref[...]) + ba_ref[...]


def _pre(x, W_emb, be, Ws, Wd, Wax, ba):
    n_blocks = _N // _BLK
    return pl.pallas_call(
        _pre_body,
        grid=(n_blocks,),
        in_specs=[
            pl.BlockSpec((_BLK, _D), lambda i: (i, 0)),
            pl.BlockSpec((_D, _H), lambda i: (0, 0)),
            pl.BlockSpec((1, _H), lambda i: (0, 0)),
            pl.BlockSpec((_D, _H), lambda i: (0, 0)),
            pl.BlockSpec((_D, _H), lambda i: (0, 0)),
            pl.BlockSpec((_D, 1), lambda i: (0, 0)),
            pl.BlockSpec((1, 1), lambda i: (0, 0)),
        ],
        out_specs=[
            pl.BlockSpec((_BLK, _H), lambda i: (i, 0)),
            pl.BlockSpec((_BLK, _H), lambda i: (i, 0)),
            pl.BlockSpec((_BLK, _H), lambda i: (i, 0)),
            pl.BlockSpec((_BLK, 1), lambda i: (i, 0)),
        ],
        out_shape=[
            jax.ShapeDtypeStruct((_N, _H), jnp.float32),
            jax.ShapeDtypeStruct((_N, _H), jnp.float32),
            jax.ShapeDtypeStruct((_N, _H), jnp.float32),
            jax.ShapeDtypeStruct((_N, 1), jnp.float32),
        ],
    )(x, W_emb, be, Ws, Wd, Wax, ba)


def _edgec_body(ea_ref, we_ref, bm_ref, c_ref):
    c_ref[...] = _f32dot(ea_ref[...], we_ref[...]) + bm_ref[...]


def _edgec(edge_attr, We, bm):
    return pl.pallas_call(
        _edgec_body,
        grid=(_E // _EBLK,),
        in_specs=[
            pl.BlockSpec((_EBLK, _DE), lambda i: (i, 0)),
            pl.BlockSpec((_DE, _H), lambda i: (0, 0)),
            pl.BlockSpec((1, _H), lambda i: (0, 0)),
        ],
        out_specs=pl.BlockSpec((_EBLK, _H), lambda i: (i, 0)),
        out_shape=jax.ShapeDtypeStruct((_E, _H), jnp.float32),
    )(edge_attr, We, bm)


def _sc_edge_body(a_hbm, b_hbm, c_hbm, src_hbm, dst_hbm, out_hbm,
                  idxs0, idxs1, idxs2,
                  idxd0a, idxd0b, idxd1a, idxd1b, idxd2a, idxd2b,
                  bufa0, bufa1, bufa2, bufb0, bufb1, bufb2,
                  bufc0, bufc1, bufc2, shared,
                  sema0, sema1, sema2, semb0, semb1, semb2,
                  semc0, semc1, semc2, semis0, semis1, semis2,
                  semid0a, semid0b, semid1a, semid1b, semid2a, semid2b,
                  semsc0, semsc1, semsc2):
    c = lax.axis_index("c")
    s = lax.axis_index("s")
    wid = c * 16 + s
    idxs = (idxs0, idxs1, idxs2)
    idxd = ((idxd0a, idxd0b), (idxd1a, idxd1b), (idxd2a, idxd2b))
    bufa = (bufa0, bufa1, bufa2)
    bufb = (bufb0, bufb1, bufb2)
    bufc = (bufc0, bufc1, bufc2)
    sema = (sema0, sema1, sema2)
    semb = (semb0, semb1, semb2)
    semc = (semc0, semc1, semc2)
    semis = (semis0, semis1, semis2)
    semid = ((semid0a, semid0b), (semid1a, semid1b), (semid2a, semid2b))
    semsc = (semsc0, semsc1, semsc2)

    zero16 = jnp.zeros((16,), jnp.float32)

    def zrow(r, carry):
        for jj in range(8):
            bufa0[r, pl.ds(jj * 16, 16)] = zero16
        return carry

    lax.fori_loop(0, _K, zrow, 0)
    row0 = s * _RPT
    for t in range(_RPT // _K):
        pltpu.sync_copy(bufa0, shared.at[pl.ds(row0 + t * _K, _K)])
    plsc.subcore_barrier()

    ebase = wid * _EPT

    def idx_issue(p, j, par):
        pltpu.async_copy(src_hbm.at[pl.ds(ebase + j * _K, _K)], idxs[p],
                         semis[p])
        pltpu.async_copy(dst_hbm.at[pl.ds(ebase + j * _K, _K)],
                         idxd[p][par], semid[p][par])

    def idx_wait(p, j, par):
        pltpu.make_async_copy(src_hbm.at[pl.ds(ebase + j * _K, _K)], idxs[p],
                              semis[p]).wait()
        pltpu.make_async_copy(dst_hbm.at[pl.ds(ebase + j * _K, _K)],
                              idxd[p][par], semid[p][par]).wait()

    def gather_issue(p, j, par):
        pltpu.async_copy(a_hbm.at[idxs[p]], bufa[p], sema[p])
        pltpu.async_copy(b_hbm.at[idxd[p][par]], bufb[p], semb[p])
        pltpu.async_copy(c_hbm.at[pl.ds(ebase + j * _K, _K)], bufc[p],
                         semc[p])

    def gather_wait(p, j, par):
        pltpu.make_async_copy(a_hbm.at[idxs[p]], bufa[p], sema[p]).wait()
        pltpu.make_async_copy(b_hbm.at[idxd[p][par]], bufb[p],
                              semb[p]).wait()
        pltpu.make_async_copy(c_hbm.at[pl.ds(ebase + j * _K, _K)], bufc[p],
                              semc[p]).wait()

    def scatter_wait(p, par):
        pltpu.make_async_copy(bufc[p], shared.at[idxd[p][par]],
                              semsc[p]).wait()

    def compute(p):
        a, b, m = bufa[p], bufb[p], bufc[p]

        def row(r, rc):
            for jj in range(8):
                sl = pl.ds(jj * 16, 16)
                m[r, sl] = jnp.maximum(a[r, sl] + b[r, sl] + m[r, sl], 0.0)
            return rc

        lax.fori_loop(0, _K, row, 0)

    def par_of(j):
        return (j // 3) % 2

    def visit(p, j, r):
        par = par_of(r)
        q = (p + 2) % 3

        gather_wait(p, j, par)
        compute(p)
        pltpu.async_copy(bufc[p], shared.at[idxd[p][par]], semsc[p],
                         add=True)

        if r != 0:
            scatter_wait(q, par_of(r - 1))
        else:
            @pl.when(j >= 1)
            def _():
                scatter_wait(q, par_of(-1))

        @pl.when(j + 2 < _NCH)
        def _():
            idx_wait(q, j + 2, par_of(r + 2))
            gather_issue(q, j + 2, par_of(r + 2))

        @pl.when(j + 3 < _NCH)
        def _():
            idx_issue(p, j + 3, 1 - par)

    idx_issue(0, 0, 0)
    idx_issue(1, 1, 0)
    idx_issue(2, 2, 0)
    idx_wait(0, 0, 0)
    gather_issue(0, 0, 0)
    idx_wait(1, 1, 0)
    gather_issue(1, 1, 0)

    def six(t, carry):
        j = 6 * t
        for r in range(6):
            visit(r % 3, j + r, r)
        return carry

    lax.fori_loop(0, _NCH // 6, six, 0)
    for jtail in range((_NCH // 6) * 6, _NCH):
        visit(jtail % 3, jtail, jtail % 6)
    scatter_wait((_NCH - 1) % 3, par_of(_NCH - 1))
    plsc.subcore_barrier()

    for t in range(_RPT // _K):
        pltpu.sync_copy(shared.at[pl.ds(row0 + t * _K, _K)],
                        out_hbm.at[c, pl.ds(row0 + t * _K, _K)])


def _sc_edge(A, B, C, src, dst):
    mesh = plsc.VectorSubcoreMesh(core_axis_name="c", subcore_axis_name="s")
    fn = functools.partial(
        pl.kernel,
        out_type=jax.ShapeDtypeStruct((2, _SCN, _H), jnp.float32),
        mesh=mesh,
        scratch_types=(
            [pltpu.VMEM((_K,), jnp.int32)] * 9
            + [pltpu.VMEM((_K, _H), jnp.float32)] * 9
            + [pltpu.VMEM_SHARED((_SCN, _H), jnp.float32)]
            + [pltpu.SemaphoreType.DMA] * 21),
    )(_sc_edge_body)
    return fn(A, B, C, src, dst)


def _gred_body(a0_ref, a1_ref, bt_ref, g_ref):
    @pl.when(pl.program_id(0) == 0)
    def _():
        g_ref[...] = jnp.zeros_like(g_ref)

    hc = a0_ref[0] + a1_ref[0]
    m = (bt_ref[...] == lax.broadcasted_iota(jnp.int32, (_BLK, _G), 1)
         ).astype(jnp.float32)
    g_ref[...] += lax.dot_general(m, hc, (((0,), (0,)), ((), ())),
                                  preferred_element_type=jnp.float32)


def _gred(aggp, bt):
    return pl.pallas_call(
        _gred_body,
        grid=(_N // _BLK,),
        in_specs=[
            pl.BlockSpec((1, _BLK, _H), lambda i: (0, i, 0)),
            pl.BlockSpec((1, _BLK, _H), lambda i: (1, i, 0)),
            pl.BlockSpec((_BLK, 1), lambda i: (i, 0)),
        ],
        out_specs=pl.BlockSpec((_G, _H), lambda i: (0, 0)),
        out_shape=jax.ShapeDtypeStruct((_G, _H), jnp.float32),
    )(aggp, aggp, bt)


def _upd_body(h_ref, l_ref, a0_ref, a1_ref, bt_ref, g_ref, wc_ref, bc_ref,
              ws_ref, wd_ref, *out_refs, want_ab):
    hc = a0_ref[0] + a1_ref[0]
    conf = jax.nn.sigmoid(_f32dot(g_ref[...], wc_ref[...]) + bc_ref[...])
    m = (bt_ref[...] == lax.broadcasted_iota(jnp.int32, (_BLK, _G), 1)
         ).astype(jnp.float32)
    cb = _f32dot(m, conf)
    left = l_ref[...]
    hn = h_ref[...] + left * cb * hc
    out_refs[0][...] = hn
    if want_ab:
        out_refs[1][...] = left * (1.0 - cb)
        out_refs[2][...] = _f32dot(hn, ws_ref[...])
        out_refs[3][...] = _f32dot(hn, wd_ref[...])


def _upd(h, left, aggp, bt, g, Wc, bc, Ws, Wd, want_ab):
    n_out = 4 if want_ab else 1
    out_shapes = [jax.ShapeDtypeStruct((_N, _H), jnp.float32),
                  jax.ShapeDtypeStruct((_N, 1), jnp.float32),
                  jax.ShapeDtypeStruct((_N, _H), jnp.float32),
                  jax.ShapeDtypeStruct((_N, _H), jnp.float32)][:n_out]
    out_specs = [pl.BlockSpec((_BLK, _H), lambda i: (i, 0)),
                 pl.BlockSpec((_BLK, 1), lambda i: (i, 0)),
                 pl.BlockSpec((_BLK, _H), lambda i: (i, 0)),
                 pl.BlockSpec((_BLK, _H), lambda i: (i, 0))][:n_out]
    return pl.pallas_call(
        functools.partial(_upd_body, want_ab=want_ab),
        grid=(_N // _BLK,),
        in_specs=[
            pl.BlockSpec((_BLK, _H), lambda i: (i, 0)),
            pl.BlockSpec((_BLK, 1), lambda i: (i, 0)),
            pl.BlockSpec((1, _BLK, _H), lambda i: (0, i, 0)),
            pl.BlockSpec((1, _BLK, _H), lambda i: (1, i, 0)),
            pl.BlockSpec((_BLK, 1), lambda i: (i, 0)),
            pl.BlockSpec((_G, _H), lambda i: (0, 0)),
            pl.BlockSpec((_H, 1), lambda i: (0, 0)),
            pl.BlockSpec((1, 1), lambda i: (0, 0)),
            pl.BlockSpec((_D, _H), lambda i: (0, 0)),
            pl.BlockSpec((_D, _H), lambda i: (0, 0)),
        ],
        out_specs=out_specs,
        out_shape=out_shapes,
    )(h, left, aggp, aggp, bt, g, Wc, bc, Ws, Wd)


def _updf_body(h_ref, l_ref, a0_ref, a1_ref, bt_ref, g_ref, wc_ref, bc_ref,
               sx_ref, wah_ref, hn_ref, sc_ref, smax_ref):
    @pl.when(pl.program_id(0) == 0)
    def _():
        smax_ref[...] = jnp.full_like(smax_ref, -1e30)

    hc = a0_ref[0] + a1_ref[0]
    conf = jax.nn.sigmoid(_f32dot(g_ref[...], wc_ref[...]) + bc_ref[...])
    mask = bt_ref[...] == lax.broadcasted_iota(jnp.int32, (_BLK, _G), 1)
    cb = _f32dot(mask.astype(jnp.float32), conf)
    hn = h_ref[...] + l_ref[...] * cb * hc
    hn_ref[...] = hn
    s = sx_ref[...] + _f32dot(hn, wah_ref[...])
    sc_ref[...] = s
    v = jnp.where(mask, s, -1e30)
    bm = jnp.max(v, axis=0)
    smax_ref[...] = jnp.maximum(smax_ref[...],
                                jnp.broadcast_to(bm[:, None], (_G, _H)))


def _updf(h, left, aggp, bt, g, Wc, bc, sx, Wah):
    return pl.pallas_call(
        _updf_body,
        grid=(_N // _BLK,),
        in_specs=[
            pl.BlockSpec((_BLK, _H), lambda i: (i, 0)),
            pl.BlockSpec((_BLK, 1), lambda i: (i, 0)),
            pl.BlockSpec((1, _BLK, _H), lambda i: (0, i, 0)),
            pl.BlockSpec((1, _BLK, _H), lambda i: (1, i, 0)),
            pl.BlockSpec((_BLK, 1), lambda i: (i, 0)),
            pl.BlockSpec((_G, _H), lambda i: (0, 0)),
            pl.BlockSpec((_H, 1), lambda i: (0, 0)),
            pl.BlockSpec((1, 1), lambda i: (0, 0)),
            pl.BlockSpec((_BLK, 1), lambda i: (i, 0)),
            pl.BlockSpec((_H, 1), lambda i: (0, 0)),
        ],
        out_specs=[
            pl.BlockSpec((_BLK, _H), lambda i: (i, 0)),
            pl.BlockSpec((_BLK, 1), lambda i: (i, 0)),
            pl.BlockSpec((_G, _H), lambda i: (0, 0)),
        ],
        out_shape=[
            jax.ShapeDtypeStruct((_N, _H), jnp.float32),
            jax.ShapeDtypeStruct((_N, 1), jnp.float32),
            jax.ShapeDtypeStruct((_G, _H), jnp.float32),
        ],
    )(h, left, aggp, aggp, bt, g, Wc, bc, sx, Wah)


def _s2_body(sc_ref, h_ref, bt_ref, smax_ref, wh_ref, bh_ref,
             out_ref, cnt_out_ref, gf_ref, den_ref, cnt_ref):
    @pl.when(pl.program_id(0) == 0)
    def _():
        gf_ref[...] = jnp.zeros_like(gf_ref)
        den_ref[...] = jnp.zeros_like(den_ref)
        cnt_ref[...] = jnp.zeros_like(cnt_ref)

    mf = (bt_ref[...] == lax.broadcasted_iota(jnp.int32, (_BLK, _G), 1)
          ).astype(jnp.float32)
    smax_col = smax_ref[...][:, 0:1]
    smax_row = _f32dot(mf, smax_col)
    e = jnp.exp(sc_ref[...] - smax_row)
    w = mf * e
    den_ref[...] += jnp.broadcast_to(jnp.sum(w, axis=0)[:, None], (_G, _H))
    cnt_ref[...] += jnp.broadcast_to(jnp.sum(mf, axis=0)[:, None], (_G, _H))
    gf_ref[...] += lax.dot_general(w, h_ref[...], (((0,), (0,)), ((), ())),
                                   preferred_element_type=jnp.float32)

    @pl.when(pl.program_id(0) == _N // _BLK - 1)
    def _():
        den = den_ref[...][:, 0:1] + 1e-16
        gfeat = gf_ref[...] / den
        out_ref[...] = _f32dot(gfeat, wh_ref[...]) + bh_ref[...]
        cnt_out_ref[...] = cnt_ref[...][:, 0:1]


def _s2(scores, h, bt, smax, Wh, bh):
    return pl.pallas_call(
        _s2_body,
        grid=(_N // _BLK,),
        in_specs=[
            pl.BlockSpec((_BLK, 1), lambda i: (i, 0)),
            pl.BlockSpec((_BLK, _H), lambda i: (i, 0)),
            pl.BlockSpec((_BLK, 1), lambda i: (i, 0)),
            pl.BlockSpec((_G, _H), lambda i: (0, 0)),
            pl.BlockSpec((_H, 1), lambda i: (0, 0)),
            pl.BlockSpec((1, 1), lambda i: (0, 0)),
        ],
        out_specs=[
            pl.BlockSpec((_G, 1), lambda i: (0, 0)),
            pl.BlockSpec((_G, 1), lambda i: (0, 0)),
        ],
        out_shape=[
            jax.ShapeDtypeStruct((_G, 1), jnp.float32),
            jax.ShapeDtypeStruct((_G, 1), jnp.float32),
        ],
        scratch_shapes=[
            pltpu.VMEM((_G, _H), jnp.float32),
            pltpu.VMEM((_G, _H), jnp.float32),
            pltpu.VMEM((_G, _H), jnp.float32),
        ],
    )(scores, h, bt, smax, Wh, bh)


def kernel(x, edge_index, edge_attr, batch, W_emb, b_emb, W_msg, b_msg,
           W_conf, b_conf, W_att, b_att, W_head, b_head):
    src = edge_index[0]
    dst = edge_index[1]
    Ws = W_msg[:_H]
    Wd = W_msg[_H:2 * _H]
    We = W_msg[2 * _H:]
    Wax = W_att[:_D]
    Wah = W_att[_D:]
    bt = batch.reshape(_N, 1)
    be = b_emb.reshape(1, _H)
    bm = b_msg.reshape(1, _H)
    bc = b_conf.reshape(1, 1)
    ba = b_att.reshape(1, 1)
    bh = b_head.reshape(1, 1)

    h, A, B, sx = _pre(x, W_emb, be, Ws, Wd, Wax, ba)
    C = _edgec(edge_attr, We, bm)
    left = jnp.ones((_N, 1), jnp.float32)

    for it in range(_ITERS):
        aggp = _sc_edge(A, B, C, src, dst)
        g = _gred(aggp, bt)
        if it < _ITERS - 1:
            h, left, A, B = _upd(h, left, aggp, bt, g, W_conf, bc, Ws, Wd,
                                 want_ab=True)
        else:
            h, scores, smax = _updf(h, left, aggp, bt, g, W_conf, bc,
                                    sx, Wah)

    out, counts = _s2(scores, h, bt, smax, W_head, bh)
    return out, counts
